# Initial kernel scaffold; baseline (speedup 1.0000x reference)
import jax
import jax.numpy as jnp
from jax.experimental import pallas as pl


def kernel(x, support0_idx, support0_val, support1_idx, support1_val, W0, W1):
    raise NotImplementedError("write your pallas kernel here")


# speedup vs baseline: 5.6159x; 5.6159x over previous
import functools

import jax
import jax.numpy as jnp
from jax import lax
from jax.experimental import pallas as pl
from jax.experimental.pallas import tpu as pltpu
from jax.experimental.pallas import tpu_sc as plsc

N = 10000
E = 320000
D = 128

NC = 2
NS = 16
EPT = E // NS
C = 80
NCHUNK = EPT // C
N_PAD = 10240
RPT = N_PAD // NS
CHB = 25
NBLK = NCHUNK // CHB


def _matmul(x, W0, W1):
    def body(x_ref, w0_ref, w1_ref, h0_ref, h1_ref):
        xb = x_ref[...]
        h0_ref[...] = jnp.dot(xb, w0_ref[...], preferred_element_type=jnp.float32)
        h1_ref[...] = jnp.dot(xb, w1_ref[...], preferred_element_type=jnp.float32)

    return pl.pallas_call(
        body,
        grid=(10,),
        in_specs=[
            pl.BlockSpec((N // 10, D), lambda i: (i, 0)),
            pl.BlockSpec((D, D), lambda i: (0, 0)),
            pl.BlockSpec((D, D), lambda i: (0, 0)),
        ],
        out_specs=[
            pl.BlockSpec((N // 10, D), lambda i: (i, 0)),
            pl.BlockSpec((N // 10, D), lambda i: (i, 0)),
        ],
        out_shape=[jax.ShapeDtypeStruct((N, D), jnp.float32)] * 2,
    )(x, W0, W1)


def _combine(p0, p1):
    def body(a_ref, b_ref, o_ref):
        o_ref[...] = jnp.maximum(a_ref[...] + b_ref[...], 0.0)

    return pl.pallas_call(
        body,
        grid=(10,),
        in_specs=[
            pl.BlockSpec((N // 10, D), lambda i: (i, 0)),
            pl.BlockSpec((N // 10, D), lambda i: (i, 0)),
        ],
        out_specs=pl.BlockSpec((N // 10, D), lambda i: (i, 0)),
        out_shape=jax.ShapeDtypeStruct((N, D), jnp.float32),
    )(p0, p1)


@functools.partial(
    pl.kernel,
    out_type=[
        jax.ShapeDtypeStruct((N_PAD, D), jnp.float32),
        jax.ShapeDtypeStruct((N_PAD, D), jnp.float32),
    ],
    mesh=plsc.VectorSubcoreMesh(core_axis_name="c", subcore_axis_name="s"),
    scratch_types=[
        pltpu.VMEM((CHB, C), jnp.int32),
        pltpu.VMEM((CHB, C), jnp.int32),
        pltpu.VMEM((CHB, C), jnp.float32),
        pltpu.VMEM((C, D), jnp.float32),
        pltpu.VMEM_SHARED((N_PAD, D), jnp.float32),
    ],
)
def _spmm_sc(h0, h1, r0, c0, v0, r1, c1, v1, p0, p1,
             rowv, colv, valv, rbuf, accum):
    c = lax.axis_index("c")
    s = lax.axis_index("s")

    def zfill(r, _):
        for f in range(D // 16):
            rbuf[r, pl.ds(f * 16, 16)] = jnp.zeros((16,), jnp.float32)
        return 0

    lax.fori_loop(0, C, zfill, 0)
    for j in range(RPT // C):
        pltpu.sync_copy(rbuf, accum.at[pl.ds(s * RPT + j * C, C)])
    plsc.subcore_barrier()

    def run_support(h, r, cc, v):
        def blk(b, _):
            pltpu.sync_copy(r.at[s, b], rowv)
            pltpu.sync_copy(cc.at[s, b], colv)
            pltpu.sync_copy(v.at[s, b], valv)

            def chunk(k, _):
                pltpu.sync_copy(h.at[colv.at[k]], rbuf)

                def escale(g, _):
                    v16 = valv[k, pl.ds(g * 16, 16)]
                    ebase = g * 16
                    for j in range(16):
                        sv = v16[j]
                        for f in range(D // 16):
                            sl = (ebase + j, pl.ds(f * 16, 16))
                            rbuf[sl] = rbuf[sl] * sv
                    return 0

                lax.fori_loop(0, C // 16, escale, 0)
                pltpu.sync_copy(rbuf, accum.at[rowv.at[k]], add=True)
                return 0

            lax.fori_loop(0, CHB, chunk, 0)
            return 0

        lax.fori_loop(0, NBLK, blk, 0)

    @pl.when(c == 0)
    def _():
        run_support(h0, r0, c0, v0)

    @pl.when(c == 1)
    def _():
        run_support(h1, r1, c1, v1)

    plsc.subcore_barrier()

    @pl.when(c == 0)
    def _():
        pltpu.sync_copy(accum.at[pl.ds(s * RPT, RPT)], p0.at[pl.ds(s * RPT, RPT)])

    @pl.when(c == 1)
    def _():
        pltpu.sync_copy(accum.at[pl.ds(s * RPT, RPT)], p1.at[pl.ds(s * RPT, RPT)])


def kernel(x, support0_idx, support0_val, support1_idx, support1_val, W0, W1):
    h0, h1 = _matmul(x.astype(jnp.float32), W0, W1)

    r0 = support0_idx[0].reshape(NS, NBLK, CHB, C)
    c0 = support0_idx[1].reshape(NS, NBLK, CHB, C)
    v0 = support0_val.reshape(NS, NBLK, CHB, C)
    r1 = support1_idx[0].reshape(NS, NBLK, CHB, C)
    c1 = support1_idx[1].reshape(NS, NBLK, CHB, C)
    v1 = support1_val.reshape(NS, NBLK, CHB, C)

    p0, p1 = _spmm_sc(h0, h1, r0, c0, v0, r1, c1, v1)
    return _combine(p0, p1)


# speedup vs baseline: 6.9696x; 1.2410x over previous
import functools

import jax
import jax.numpy as jnp
from jax import lax
from jax.experimental import pallas as pl
from jax.experimental.pallas import tpu as pltpu
from jax.experimental.pallas import tpu_sc as plsc

N = 10000
E = 320000
D = 128

NC = 2
NS = 16
EPT = E // NS
C = 40
NCHUNK = EPT // C
N_PAD = 10240
RPT = N_PAD // NS
CHB = 50
NBLK = NCHUNK // CHB


def _matmul(x, W0, W1):
    def body(x_ref, w0_ref, w1_ref, h0_ref, h1_ref):
        xb = x_ref[...]
        h0_ref[...] = jnp.dot(xb, w0_ref[...], preferred_element_type=jnp.float32)
        h1_ref[...] = jnp.dot(xb, w1_ref[...], preferred_element_type=jnp.float32)

    return pl.pallas_call(
        body,
        grid=(10,),
        in_specs=[
            pl.BlockSpec((N // 10, D), lambda i: (i, 0)),
            pl.BlockSpec((D, D), lambda i: (0, 0)),
            pl.BlockSpec((D, D), lambda i: (0, 0)),
        ],
        out_specs=[
            pl.BlockSpec((N // 10, D), lambda i: (i, 0)),
            pl.BlockSpec((N // 10, D), lambda i: (i, 0)),
        ],
        out_shape=[jax.ShapeDtypeStruct((N, D), jnp.float32)] * 2,
    )(x, W0, W1)


def _combine(p0, p1):
    def body(a_ref, b_ref, o_ref):
        o_ref[...] = jnp.maximum(a_ref[...] + b_ref[...], 0.0)

    return pl.pallas_call(
        body,
        grid=(10,),
        in_specs=[
            pl.BlockSpec((N // 10, D), lambda i: (i, 0)),
            pl.BlockSpec((N // 10, D), lambda i: (i, 0)),
        ],
        out_specs=pl.BlockSpec((N // 10, D), lambda i: (i, 0)),
        out_shape=jax.ShapeDtypeStruct((N, D), jnp.float32),
    )(p0, p1)


@functools.partial(
    pl.kernel,
    out_type=[
        jax.ShapeDtypeStruct((N_PAD, D), jnp.float32),
        jax.ShapeDtypeStruct((N_PAD, D), jnp.float32),
    ],
    mesh=plsc.VectorSubcoreMesh(core_axis_name="c", subcore_axis_name="s"),
    scratch_types=[
        pltpu.VMEM((CHB, C), jnp.int32),
        pltpu.VMEM((CHB, C), jnp.int32),
        pltpu.VMEM((CHB, C), jnp.float32),
        pltpu.VMEM((C, D), jnp.float32),
        pltpu.VMEM((C, D), jnp.float32),
        pltpu.VMEM_SHARED((N_PAD, D), jnp.float32),
        pltpu.SemaphoreType.DMA,
        pltpu.SemaphoreType.DMA,
        pltpu.SemaphoreType.DMA,
        pltpu.SemaphoreType.DMA,
    ],
)
def _spmm_sc(h0, h1, r0, c0, v0, r1, c1, v1, p0, p1,
             rowv, colv, valv, rbufa, rbufb, accum,
             gsema, gsemb, ssema, ssemb):
    c = lax.axis_index("c")
    s = lax.axis_index("s")

    def zfill(r, _):
        for f in range(D // 16):
            rbufa[r, pl.ds(f * 16, 16)] = jnp.zeros((16,), jnp.float32)
        return 0

    lax.fori_loop(0, C, zfill, 0)
    for j in range(RPT // C):
        pltpu.sync_copy(rbufa, accum.at[pl.ds(s * RPT + j * C, C)])
    plsc.subcore_barrier()

    def scale(buf, k):
        def group(g, _):
            v16 = valv[k, pl.ds(g * 16, 16)]
            ebase = g * 16
            for j in range(16):
                sv = v16[j]
                for f in range(D // 16):
                    sl = (ebase + j, pl.ds(f * 16, 16))
                    buf[sl] = buf[sl] * sv
            return 0

        lax.fori_loop(0, C // 16, group, 0)
        v16 = valv[k, pl.ds(C - 16, 16)]
        for j in range(32 - (C - 16), 16):
            sv = v16[j]
            for f in range(D // 16):
                sl = ((C - 16) + j, pl.ds(f * 16, 16))
                buf[sl] = buf[sl] * sv

    def run_support(h, r, cc, v):
        def blk(b, _):
            pltpu.sync_copy(r.at[s, b], rowv)
            pltpu.sync_copy(cc.at[s, b], colv)
            pltpu.sync_copy(v.at[s, b], valv)
            pltpu.async_copy(h.at[colv.at[0]], rbufa, gsema)

            def pair(p, _):
                k0 = 2 * p
                k1 = k0 + 1

                @pl.when(p > 0)
                def _():
                    pltpu.make_async_copy(
                        rbufb, accum.at[rowv.at[k1 - 2]], ssemb).wait()

                pltpu.async_copy(h.at[colv.at[k1]], rbufb, gsemb)
                pltpu.make_async_copy(h.at[colv.at[k0]], rbufa, gsema).wait()
                scale(rbufa, k0)
                pltpu.async_copy(rbufa, accum.at[rowv.at[k0]], ssema, add=True)
                pltpu.make_async_copy(h.at[colv.at[k1]], rbufb, gsemb).wait()
                scale(rbufb, k1)
                pltpu.async_copy(rbufb, accum.at[rowv.at[k1]], ssemb, add=True)
                pltpu.make_async_copy(
                    rbufa, accum.at[rowv.at[k0]], ssema).wait()

                @pl.when(k0 + 2 < CHB)
                def _():
                    pltpu.async_copy(h.at[colv.at[k0 + 2]], rbufa, gsema)

                return 0

            lax.fori_loop(0, CHB // 2, pair, 0)
            pltpu.make_async_copy(
                rbufb, accum.at[rowv.at[CHB - 1]], ssemb).wait()
            return 0

        lax.fori_loop(0, NBLK, blk, 0)

    @pl.when(c == 0)
    def _():
        run_support(h0, r0, c0, v0)

    @pl.when(c == 1)
    def _():
        run_support(h1, r1, c1, v1)

    plsc.subcore_barrier()

    @pl.when(c == 0)
    def _():
        pltpu.sync_copy(accum.at[pl.ds(s * RPT, RPT)], p0.at[pl.ds(s * RPT, RPT)])

    @pl.when(c == 1)
    def _():
        pltpu.sync_copy(accum.at[pl.ds(s * RPT, RPT)], p1.at[pl.ds(s * RPT, RPT)])


def kernel(x, support0_idx, support0_val, support1_idx, support1_val, W0, W1):
    h0, h1 = _matmul(x.astype(jnp.float32), W0, W1)

    r0 = support0_idx[0].reshape(NS, NBLK, CHB, C)
    c0 = support0_idx[1].reshape(NS, NBLK, CHB, C)
    v0 = support0_val.reshape(NS, NBLK, CHB, C)
    r1 = support1_idx[0].reshape(NS, NBLK, CHB, C)
    c1 = support1_idx[1].reshape(NS, NBLK, CHB, C)
    v1 = support1_val.reshape(NS, NBLK, CHB, C)

    p0, p1 = _spmm_sc(h0, h1, r0, c0, v0, r1, c1, v1)
    return _combine(p0, p1)


# speedup vs baseline: 7.4017x; 1.0620x over previous
import functools

import jax
import jax.numpy as jnp
from jax import lax
from jax.experimental import pallas as pl
from jax.experimental.pallas import tpu as pltpu
from jax.experimental.pallas import tpu_sc as plsc

N = 10000
E = 320000
D = 128

NC = 2
NS = 16
EPT = E // NS
C = 40
NCHUNK = EPT // C
N_PAD = 10240
RPT = N_PAD // NS
CHB = 20
NBLK = NCHUNK // CHB


def _matmul(x, W0, W1):
    def body(x_ref, w0_ref, w1_ref, h0_ref, h1_ref):
        xb = x_ref[...]
        h0_ref[...] = jnp.dot(xb, w0_ref[...], preferred_element_type=jnp.float32)
        h1_ref[...] = jnp.dot(xb, w1_ref[...], preferred_element_type=jnp.float32)

    return pl.pallas_call(
        body,
        grid=(10,),
        in_specs=[
            pl.BlockSpec((N // 10, D), lambda i: (i, 0)),
            pl.BlockSpec((D, D), lambda i: (0, 0)),
            pl.BlockSpec((D, D), lambda i: (0, 0)),
        ],
        out_specs=[
            pl.BlockSpec((N // 10, D), lambda i: (i, 0)),
            pl.BlockSpec((N // 10, D), lambda i: (i, 0)),
        ],
        out_shape=[jax.ShapeDtypeStruct((N, D), jnp.float32)] * 2,
    )(x, W0, W1)


def _combine(p0, p1):
    def body(a_ref, b_ref, o_ref):
        o_ref[...] = jnp.maximum(a_ref[...] + b_ref[...], 0.0)

    return pl.pallas_call(
        body,
        grid=(10,),
        in_specs=[
            pl.BlockSpec((N // 10, D), lambda i: (i, 0)),
            pl.BlockSpec((N // 10, D), lambda i: (i, 0)),
        ],
        out_specs=pl.BlockSpec((N // 10, D), lambda i: (i, 0)),
        out_shape=jax.ShapeDtypeStruct((N, D), jnp.float32),
    )(p0, p1)


@functools.partial(
    pl.kernel,
    out_type=[
        jax.ShapeDtypeStruct((N_PAD, D), jnp.float32),
        jax.ShapeDtypeStruct((N_PAD, D), jnp.float32),
    ],
    mesh=plsc.VectorSubcoreMesh(core_axis_name="c", subcore_axis_name="s"),
    scratch_types=[
        pltpu.VMEM((CHB, C), jnp.int32),
        pltpu.VMEM((CHB, C), jnp.int32),
        pltpu.VMEM((CHB, C), jnp.float32),
        [pltpu.VMEM((C, D), jnp.float32)] * 4,
        pltpu.VMEM_SHARED((N_PAD, D), jnp.float32),
        [pltpu.SemaphoreType.DMA] * 4,
        [pltpu.SemaphoreType.DMA] * 4,
    ],
)
def _spmm_sc(h0, h1, r0, c0, v0, r1, c1, v1, p0, p1,
             rowv, colv, valv, rbufs, accum, gsems, ssems):
    c = lax.axis_index("c")
    s = lax.axis_index("s")

    def zfill(r, _):
        for f in range(D // 16):
            rbufs[0][r, pl.ds(f * 16, 16)] = jnp.zeros((16,), jnp.float32)
        return 0

    lax.fori_loop(0, C, zfill, 0)
    for j in range(RPT // C):
        pltpu.sync_copy(rbufs[0], accum.at[pl.ds(s * RPT + j * C, C)])
    plsc.subcore_barrier()

    def scale(buf, k):
        def group(g, _):
            v16 = valv[k, pl.ds(g * 16, 16)]
            ebase = g * 16
            for j in range(16):
                sv = v16[j]
                for f in range(D // 16):
                    sl = (ebase + j, pl.ds(f * 16, 16))
                    buf[sl] = buf[sl] * sv
            return 0

        lax.fori_loop(0, C // 16, group, 0)
        v16 = valv[k, pl.ds(C - 16, 16)]
        for j in range(32 - (C - 16), 16):
            sv = v16[j]
            for f in range(D // 16):
                sl = ((C - 16) + j, pl.ds(f * 16, 16))
                buf[sl] = buf[sl] * sv

    def run_support(h, r, cc, v):
        def blk(b, _):
            pltpu.sync_copy(r.at[s, b], rowv)
            pltpu.sync_copy(cc.at[s, b], colv)
            pltpu.sync_copy(v.at[s, b], valv)
            pltpu.async_copy(h.at[colv.at[0]], rbufs[0], gsems[0])
            pltpu.async_copy(h.at[colv.at[1]], rbufs[1], gsems[1])

            def quad(q, _):
                for i in range(4):
                    k = 4 * q + i
                    bf = i
                    nb = (i + 2) % 4
                    pltpu.make_async_copy(
                        h.at[colv.at[k]], rbufs[bf], gsems[bf]).wait()
                    scale(rbufs[bf], k)
                    pltpu.async_copy(
                        rbufs[bf], accum.at[rowv.at[k]], ssems[bf], add=True)

                    @pl.when(k + 2 < CHB)
                    def _():
                        @pl.when(k >= 2)
                        def _():
                            pltpu.make_async_copy(
                                rbufs[nb], accum.at[rowv.at[k - 2]],
                                ssems[nb]).wait()

                        pltpu.async_copy(
                            h.at[colv.at[k + 2]], rbufs[nb], gsems[nb])

                return 0

            lax.fori_loop(0, CHB // 4, quad, 0)
            for k in range(CHB - 4, CHB):
                pltpu.make_async_copy(
                    rbufs[k % 4], accum.at[rowv.at[k]],
                    ssems[k % 4]).wait()
            return 0

        lax.fori_loop(0, NBLK, blk, 0)

    @pl.when(c == 0)
    def _():
        run_support(h0, r0, c0, v0)

    @pl.when(c == 1)
    def _():
        run_support(h1, r1, c1, v1)

    plsc.subcore_barrier()

    @pl.when(c == 0)
    def _():
        pltpu.sync_copy(accum.at[pl.ds(s * RPT, RPT)], p0.at[pl.ds(s * RPT, RPT)])

    @pl.when(c == 1)
    def _():
        pltpu.sync_copy(accum.at[pl.ds(s * RPT, RPT)], p1.at[pl.ds(s * RPT, RPT)])


def kernel(x, support0_idx, support0_val, support1_idx, support1_val, W0, W1):
    h0, h1 = _matmul(x.astype(jnp.float32), W0, W1)

    r0 = support0_idx[0].reshape(NS, NBLK, CHB, C)
    c0 = support0_idx[1].reshape(NS, NBLK, CHB, C)
    v0 = support0_val.reshape(NS, NBLK, CHB, C)
    r1 = support1_idx[0].reshape(NS, NBLK, CHB, C)
    c1 = support1_idx[1].reshape(NS, NBLK, CHB, C)
    v1 = support1_val.reshape(NS, NBLK, CHB, C)

    p0, p1 = _spmm_sc(h0, h1, r0, c0, v0, r1, c1, v1)
    return _combine(p0, p1)
